# baseline (speedup 1.0000x reference)
import functools

import jax
import jax.numpy as jnp
from jax import lax
from jax.experimental import pallas as pl
from jax.experimental.pallas import tpu as pltpu
from jax.experimental.pallas import tpu_sc as plsc

N = 10000
E = 320000
D = 128

NC = 2
NS = 16
NW = NC * NS
EPW = E // NW
K = 80
NCHUNK = EPW // K
NSLOT = 4
NIB = 8
NPAD = 10112
RPS = NPAD // NS


def _mlp_body(x_ref, w0t_ref, b0_ref, w1t_ref, b1_ref, o_ref):
    h = jnp.dot(x_ref[...], w0t_ref[...], preferred_element_type=jnp.float32)
    h = h + b0_ref[...]
    h = jnp.dot(h, w1t_ref[...], preferred_element_type=jnp.float32)
    h = h + b1_ref[...]
    o_ref[...] = jnp.where(h > 0, h, jnp.exp(jnp.minimum(h, 0.0)) - 1.0)


def _mlp(x, w0t, b0, w1t, b1):
    BN = 1000
    return pl.pallas_call(
        _mlp_body,
        grid=(N // BN,),
        in_specs=[
            pl.BlockSpec((BN, D), lambda i: (i, 0)),
            pl.BlockSpec((D, D), lambda i: (0, 0)),
            pl.BlockSpec((1, D), lambda i: (0, 0)),
            pl.BlockSpec((D, D), lambda i: (0, 0)),
            pl.BlockSpec((1, D), lambda i: (0, 0)),
        ],
        out_specs=pl.BlockSpec((BN, D), lambda i: (i, 0)),
        out_shape=jax.ShapeDtypeStruct((N, D), jnp.float32),
    )(x, w0t, b0, w1t, b1)


def _sc_agg(emb, idx, zeros):
    mesh = plsc.VectorSubcoreMesh(core_axis_name="c", subcore_axis_name="s")

    @functools.partial(
        pl.kernel,
        out_type=jax.ShapeDtypeStruct((NC, NPAD, D), jnp.float32),
        mesh=mesh,
        scratch_types=[
            pltpu.VMEM((NIB, 2, K), jnp.int32),
            pltpu.VMEM((NSLOT, K, D), jnp.float32),
            pltpu.VMEM_SHARED((NPAD, D), jnp.float32),
        ] + [pltpu.SemaphoreType.DMA] * (NIB + 2 * NSLOT),
    )
    def k(emb_hbm, idx_hbm, z_hbm, out_hbm, ib, rows, acc, *sems):
        si = sems[:NIB]
        sg = sems[NIB:NIB + NSLOT]
        ss = sems[NIB + NSLOT:]
        c = lax.axis_index("c")
        s = lax.axis_index("s")
        wid = c * NS + s

        for q in range(NSLOT):
            pltpu.async_copy(idx_hbm.at[wid, q], ib.at[q], si[q])
        pltpu.sync_copy(z_hbm, acc.at[pl.ds(s * RPS, RPS)])
        plsc.subcore_barrier()

        for r in range(NSLOT - 1):
            pltpu.make_async_copy(idx_hbm.at[wid, r], ib.at[r], si[r]).wait()
            pltpu.async_copy(emb_hbm.at[ib.at[r, 0]], rows.at[r], sg[r])

        def turn(ch, t, guard):
            r = t % NSLOT
            r3 = (r + NSLOT - 1) % NSLOT
            q3 = (t + NSLOT - 1) % NIB
            q4 = (t + NSLOT) % NIB
            qm1 = (t + NIB - 1) % NIB
            pltpu.make_async_copy(
                emb_hbm.at[ib.at[t, 0]], rows.at[r], sg[r]).wait()

            def wait_prev_scatter():
                pltpu.make_async_copy(
                    rows.at[r3], acc.at[ib.at[qm1, 1]], ss[r3]).wait()

            if guard is None:
                wait_prev_scatter()
            elif guard is not False:
                pl.when(guard)(wait_prev_scatter)
            pltpu.async_copy(rows.at[r], acc.at[ib.at[t, 1]], ss[r], add=True)

            def prefetch():
                pltpu.async_copy(
                    idx_hbm.at[wid, ch + NSLOT], ib.at[q4], si[q4])

            def launch_gather():
                pltpu.make_async_copy(
                    idx_hbm.at[wid, ch + NSLOT - 1], ib.at[q3], si[q3]).wait()
                pltpu.async_copy(emb_hbm.at[ib.at[q3, 0]], rows.at[r3], sg[r3])

            if isinstance(ch, int):
                if ch + NSLOT < NCHUNK:
                    prefetch()
                if ch + NSLOT - 1 < NCHUNK:
                    launch_gather()
            else:
                pl.when(ch + NSLOT < NCHUNK)(prefetch)
                pl.when(ch + NSLOT - 1 < NCHUNK)(launch_gather)

        @pl.loop(0, NCHUNK // NIB)
        def _(j):
            for t in range(NIB):
                turn(NIB * j + t, t, (j > 0) if t == 0 else None)

        for t in range(NCHUNK % NIB):
            turn((NCHUNK // NIB) * NIB + t, t, None)

        lc = NCHUNK - 1
        pltpu.make_async_copy(
            rows.at[lc % NSLOT], acc.at[ib.at[lc % NIB, 1]],
            ss[lc % NSLOT]).wait()

        plsc.subcore_barrier()
        pltpu.sync_copy(
            acc.at[pl.ds(s * RPS, RPS)],
            out_hbm.at[c, pl.ds(s * RPS, RPS)],
        )

    return k(emb, idx, zeros)


def _add_body(p_ref, o_ref):
    o_ref[...] = p_ref[0] + p_ref[1]


def _partial_add(p):
    BN = 1000
    return pl.pallas_call(
        _add_body,
        grid=(N // BN,),
        in_specs=[pl.BlockSpec((NC, BN, D), lambda i: (0, i, 0))],
        out_specs=pl.BlockSpec((BN, D), lambda i: (i, 0)),
        out_shape=jax.ShapeDtypeStruct((N, D), jnp.float32),
    )(p)


def kernel(graph_embedding, edge_index, W0, b0, W1, b1):
    x = graph_embedding.astype(jnp.float32)
    emb = _mlp(x, W0.T, b0.reshape(1, D), W1.T, b1.reshape(1, D))
    idx = edge_index.astype(jnp.int32).reshape(2, NW, NCHUNK, K)
    idx = jnp.transpose(idx, (1, 2, 0, 3))
    zeros = jnp.zeros((RPS, D), jnp.float32)
    partials = _sc_agg(emb, idx, zeros)
    return _partial_add(partials)


# speedup vs baseline: 1.0107x; 1.0107x over previous
import functools

import jax
import jax.numpy as jnp
from jax import lax
from jax.experimental import pallas as pl
from jax.experimental.pallas import tpu as pltpu
from jax.experimental.pallas import tpu_sc as plsc

N = 10000
E = 320000
D = 128

NC = 2
NS = 16
NW = NC * NS
EPW = E // NW
K = 80
NCHUNK = EPW // K
NSLOT = 4
NPAD = 10112
RPS = NPAD // NS


def _mlp_body(x_ref, w0t_ref, b0_ref, w1t_ref, b1_ref, o_ref):
    h = jnp.dot(x_ref[...], w0t_ref[...], preferred_element_type=jnp.float32)
    h = h + b0_ref[...]
    h = jnp.dot(h, w1t_ref[...], preferred_element_type=jnp.float32)
    h = h + b1_ref[...]
    o_ref[...] = jnp.where(h > 0, h, jnp.exp(jnp.minimum(h, 0.0)) - 1.0)


def _mlp(x, w0t, b0, w1t, b1):
    BN = 1000
    return pl.pallas_call(
        _mlp_body,
        grid=(N // BN,),
        in_specs=[
            pl.BlockSpec((BN, D), lambda i: (i, 0)),
            pl.BlockSpec((D, D), lambda i: (0, 0)),
            pl.BlockSpec((1, D), lambda i: (0, 0)),
            pl.BlockSpec((D, D), lambda i: (0, 0)),
            pl.BlockSpec((1, D), lambda i: (0, 0)),
        ],
        out_specs=pl.BlockSpec((BN, D), lambda i: (i, 0)),
        out_shape=jax.ShapeDtypeStruct((N, D), jnp.float32),
    )(x, w0t, b0, w1t, b1)


def _sc_agg(emb, idx, zeros):
    mesh = plsc.VectorSubcoreMesh(core_axis_name="c", subcore_axis_name="s")

    @functools.partial(
        pl.kernel,
        out_type=jax.ShapeDtypeStruct((NC, NPAD, D), jnp.float32),
        mesh=mesh,
        scratch_types=[
            pltpu.VMEM((NSLOT, 2, K), jnp.int32),
            pltpu.VMEM((NSLOT, K, D), jnp.float32),
            pltpu.VMEM_SHARED((NPAD, D), jnp.float32),
        ] + [pltpu.SemaphoreType.DMA] * (2 * NSLOT),
    )
    def k(emb_hbm, idx_hbm, z_hbm, out_hbm, ib, rows, acc, *sems):
        si = sems[:NSLOT]
        sg = sems[NSLOT:]
        c = lax.axis_index("c")
        s = lax.axis_index("s")
        wid = c * NS + s

        for r in range(NSLOT):
            pltpu.async_copy(idx_hbm.at[wid, r], ib.at[r], si[r])
        for r in range(NSLOT - 1):
            pltpu.make_async_copy(idx_hbm.at[wid, r], ib.at[r], si[r]).wait()
            pltpu.async_copy(emb_hbm.at[ib.at[r, 0]], rows.at[r], sg[r])
        pltpu.sync_copy(z_hbm, acc.at[pl.ds(s * RPS, RPS)])
        plsc.subcore_barrier()

        @pl.loop(0, NCHUNK // NSLOT)
        def _(j):
            c0 = NSLOT * j
            for r in range(NSLOT):
                ch = c0 + r
                r3 = (r + NSLOT - 1) % NSLOT
                pltpu.make_async_copy(
                    emb_hbm.at[ib.at[r, 0]], rows.at[r], sg[r]).wait()
                pltpu.sync_copy(rows.at[r], acc.at[ib.at[r, 1]], add=True)

                @pl.when(ch + NSLOT < NCHUNK)
                def _():
                    pltpu.async_copy(idx_hbm.at[wid, ch + NSLOT], ib.at[r], si[r])

                @pl.when(ch + NSLOT - 1 < NCHUNK)
                def _():
                    pltpu.make_async_copy(
                        idx_hbm.at[wid, ch + NSLOT - 1], ib.at[r3], si[r3]).wait()
                    pltpu.async_copy(emb_hbm.at[ib.at[r3, 0]], rows.at[r3], sg[r3])

        for r in range(NCHUNK % NSLOT):
            pltpu.make_async_copy(
                emb_hbm.at[ib.at[r, 0]], rows.at[r], sg[r]).wait()
            pltpu.sync_copy(rows.at[r], acc.at[ib.at[r, 1]], add=True)

        plsc.subcore_barrier()
        pltpu.sync_copy(
            acc.at[pl.ds(s * RPS, RPS)],
            out_hbm.at[c, pl.ds(s * RPS, RPS)],
        )

    return k(emb, idx, zeros)


def _add_body(p_ref, o_ref):
    o_ref[...] = p_ref[0] + p_ref[1]


def _partial_add(p):
    BN = 1000
    return pl.pallas_call(
        _add_body,
        grid=(N // BN,),
        in_specs=[pl.BlockSpec((NC, BN, D), lambda i: (0, i, 0))],
        out_specs=pl.BlockSpec((BN, D), lambda i: (i, 0)),
        out_shape=jax.ShapeDtypeStruct((N, D), jnp.float32),
    )(p)


def kernel(graph_embedding, edge_index, W0, b0, W1, b1):
    x = graph_embedding.astype(jnp.float32)
    emb = _mlp(x, W0.T, b0.reshape(1, D), W1.T, b1.reshape(1, D))
    idx = edge_index.astype(jnp.int32).reshape(2, NW, NCHUNK, K)
    idx = jnp.transpose(idx, (1, 2, 0, 3))
    zeros = jnp.zeros((RPS, D), jnp.float32)
    partials = _sc_agg(emb, idx, zeros)
    return _partial_add(partials)
